# baseline (speedup 1.0000x reference)
import functools

import jax
import jax.numpy as jnp
from jax import lax
from jax.experimental import pallas as pl
from jax.experimental.pallas import tpu as pltpu
from jax.experimental.pallas import tpu_sc as plsc

NC = 2
NS = 16
CHUNK = 100
NBUF = 5


def _make_sc_aggregate(n_pad, dh, n_chunks, parity):
    rows_per_sub = n_pad // NS

    @functools.partial(
        pl.kernel,
        out_type=(
            jax.ShapeDtypeStruct((n_pad, dh), jnp.float32),
            jax.ShapeDtypeStruct((n_pad, 16), jnp.float32),
        ),
        mesh=plsc.VectorSubcoreMesh(
            core_axis_name="c", subcore_axis_name="s",
            num_cores=1, num_subcores=NS),
        scratch_types=[
            pltpu.VMEM((n_chunks, CHUNK), jnp.int32),
            pltpu.VMEM((n_chunks, CHUNK), jnp.int32),
            [pltpu.VMEM((CHUNK, dh), jnp.float32)] * NBUF,
            pltpu.VMEM((CHUNK, 16), jnp.float32),
            pltpu.VMEM_SHARED((n_pad, dh), jnp.float32),
            pltpu.VMEM_SHARED((n_pad, 16), jnp.float32),
            pltpu.SemaphoreType.DMA,
            pltpu.SemaphoreType.DMA,
            pltpu.SemaphoreType.DMA,
        ],
        compiler_params=pltpu.CompilerParams(use_tc_tiling_on_sc=False),
    )
    def sc_aggregate(x_hbm, src_hbm, dst_hbm, zag_hbm, zcnt_hbm,
                     ones_hbm, agg_out, cnt_out,
                     src_v, dst_v, rows_ring, ones_v, agg_sh, cnt_sh,
                     sem_g, sem_a, sem_c):
        s = lax.axis_index("s")
        rslice = pl.ds(s * rows_per_sub, rows_per_sub)

        pltpu.sync_copy(zag_hbm, agg_sh.at[rslice])
        pltpu.sync_copy(zcnt_hbm, cnt_sh.at[rslice])
        pltpu.sync_copy(ones_hbm, ones_v)
        pltpu.sync_copy(src_hbm.at[s], src_v)
        pltpu.sync_copy(dst_hbm.at[s], dst_v)
        plsc.subcore_barrier()

        def start_gather(j, buf):
            pltpu.async_copy(x_hbm.at[src_v.at[j]], buf, sem_g)

        def wait_gather(buf):
            pltpu.make_async_copy(x_hbm.at[src_v.at[0]], buf, sem_g).wait()

        def wait_scatter(buf):
            pltpu.make_async_copy(x_hbm.at[src_v.at[0]], buf, sem_a).wait()

        def half_step(j, buf_cur, buf_next):
            wait_gather(buf_cur)
            pltpu.async_copy(buf_cur, agg_sh.at[dst_v.at[j]], sem_a, add=True)

            @pl.when(lax.rem(j, 2) == parity)
            def _():
                pltpu.async_copy(ones_v, cnt_sh.at[dst_v.at[j]], sem_c,
                                 add=True)

            @pl.when(j >= 1)
            def _():
                wait_scatter(buf_next)

            @pl.when(j + NBUF - 1 < n_chunks)
            def _():
                start_gather(j + NBUF - 1, buf_next)

        for b in range(NBUF - 1):
            start_gather(b, rows_ring[b])

        def chunk_step(jg, carry):
            for b in range(NBUF):
                half_step(NBUF * jg + b, rows_ring[b],
                          rows_ring[(b + NBUF - 1) % NBUF])
            return carry

        lax.fori_loop(0, n_chunks // NBUF, chunk_step, 0)

        wait_scatter(rows_ring[0])

        def drain_cnt(i, carry):
            pltpu.make_async_copy(ones_hbm, ones_v, sem_c).wait()
            return carry

        lax.fori_loop(0, n_chunks // 2, drain_cnt, 0)
        plsc.subcore_barrier()

        pltpu.sync_copy(agg_sh.at[rslice], agg_out.at[rslice])
        pltpu.sync_copy(cnt_sh.at[rslice], cnt_out.at[rslice])

    return sc_aggregate


def _tc_combine_body(p0, p1, c0, c1, x, wl, wr, b, o):
    dh = p0.shape[1]
    cnt = c0[:, 0:1] + c1[:, 0:1]
    recip = 1.0 / jnp.maximum(cnt, 1.0)
    mean_l = p0[...] * recip
    mean_r = p1[...] * recip
    out = (jnp.dot(mean_l, wl[:dh, :], preferred_element_type=jnp.float32)
           + jnp.dot(mean_r, wl[dh:, :], preferred_element_type=jnp.float32)
           + jnp.dot(x[...], wr[...], preferred_element_type=jnp.float32)
           + b[...])
    o[...] = jnp.maximum(out, 0.0)


def _tc_combine(p0, p1, c0, c1, x, wl, wr, b):
    n, d = x.shape
    dh = d // 2
    blk = 1000
    grid = n // blk
    return pl.pallas_call(
        _tc_combine_body,
        grid=(grid,),
        in_specs=[
            pl.BlockSpec((blk, dh), lambda i: (i, 0)),
            pl.BlockSpec((blk, dh), lambda i: (i, 0)),
            pl.BlockSpec((blk, 16), lambda i: (i, 0)),
            pl.BlockSpec((blk, 16), lambda i: (i, 0)),
            pl.BlockSpec((blk, d), lambda i: (i, 0)),
            pl.BlockSpec((d, d), lambda i: (0, 0)),
            pl.BlockSpec((d, d), lambda i: (0, 0)),
            pl.BlockSpec((1, d), lambda i: (0, 0)),
        ],
        out_specs=pl.BlockSpec((blk, d), lambda i: (i, 0)),
        out_shape=jax.ShapeDtypeStruct((n, d), jnp.float32),
    )(p0, p1, c0, c1, x, wl, wr, b)


def kernel(x, edge_index, W_l, W_r, b):
    n, d = x.shape
    dh = d // 2
    n_edges = edge_index.shape[1]
    edges_per_sub = n_edges // NS
    n_chunks = edges_per_sub // CHUNK
    align = NS * 8
    n_pad = ((n + align - 1) // align) * align
    rows_per_sub = n_pad // NS

    ei = edge_index.astype(jnp.int32)
    src = ei[0].reshape(NS, n_chunks, CHUNK)
    dst = ei[1].reshape(NS, n_chunks, CHUNK)
    xl = x[:, :dh]
    xr = x[:, dh:]
    zag = jnp.zeros((rows_per_sub, dh), jnp.float32)
    zcnt = jnp.zeros((rows_per_sub, 16), jnp.float32)
    ones = jnp.ones((CHUNK, 16), jnp.float32)

    agg0, cnt0 = _make_sc_aggregate(n_pad, dh, n_chunks, 0)(
        xl, src, dst, zag, zcnt, ones)
    agg1, cnt1 = _make_sc_aggregate(n_pad, dh, n_chunks, 1)(
        xr, src, dst, zag, zcnt, ones)
    return _tc_combine(agg0, agg1, cnt0, cnt1, x, W_l, W_r, b.reshape(1, d))


# speedup vs baseline: 1.4643x; 1.4643x over previous
import functools

import jax
import jax.numpy as jnp
from jax import lax
from jax.experimental import pallas as pl
from jax.experimental.pallas import tpu as pltpu
from jax.experimental.pallas import tpu_sc as plsc

NC = 2
NS = 16
CHUNK = 100
NBUF = 5


def _make_sc_aggregate(n_pad, dh, n_chunks):
    rows_per_sub = n_pad // NS

    @functools.partial(
        pl.kernel,
        out_type=(
            jax.ShapeDtypeStruct((n_pad, dh), jnp.float32),
            jax.ShapeDtypeStruct((n_pad, dh), jnp.float32),
            jax.ShapeDtypeStruct((n_pad, 16), jnp.float32),
            jax.ShapeDtypeStruct((n_pad, 16), jnp.float32),
        ),
        mesh=plsc.VectorSubcoreMesh(
            core_axis_name="c", subcore_axis_name="s",
            num_cores=NC, num_subcores=NS),
        scratch_types=[
            pltpu.VMEM((n_chunks, CHUNK), jnp.int32),
            pltpu.VMEM((n_chunks, CHUNK), jnp.int32),
            [pltpu.VMEM((CHUNK, dh), jnp.float32)] * NBUF,
            pltpu.VMEM((CHUNK, 16), jnp.float32),
            pltpu.VMEM_SHARED((n_pad, dh), jnp.float32),
            pltpu.VMEM_SHARED((n_pad, 16), jnp.float32),
            pltpu.SemaphoreType.DMA,
            pltpu.SemaphoreType.DMA,
            pltpu.SemaphoreType.DMA,
        ],
        compiler_params=pltpu.CompilerParams(use_tc_tiling_on_sc=False),
    )
    def sc_aggregate(xl_hbm, xr_hbm, src_hbm, dst_hbm, zag_hbm, zcnt_hbm,
                     ones_hbm, agg0_out, agg1_out, cnt0_out, cnt1_out,
                     src_v, dst_v, rows_ring, ones_v, agg_sh, cnt_sh,
                     sem_g, sem_a, sem_c):
        c = lax.axis_index("c")
        s = lax.axis_index("s")
        rslice = pl.ds(s * rows_per_sub, rows_per_sub)

        pltpu.sync_copy(zag_hbm, agg_sh.at[rslice])
        pltpu.sync_copy(zcnt_hbm, cnt_sh.at[rslice])
        pltpu.sync_copy(ones_hbm, ones_v)
        pltpu.sync_copy(src_hbm.at[s], src_v)
        pltpu.sync_copy(dst_hbm.at[s], dst_v)
        plsc.subcore_barrier()

        def start_gather(j, buf):
            @pl.when(c == 0)
            def _():
                pltpu.async_copy(xl_hbm.at[src_v.at[j]], buf, sem_g)

            @pl.when(c == 1)
            def _():
                pltpu.async_copy(xr_hbm.at[src_v.at[j]], buf, sem_g)

        def wait_gather(buf):
            pltpu.make_async_copy(xl_hbm.at[src_v.at[0]], buf, sem_g).wait()

        def wait_scatter(buf):
            pltpu.make_async_copy(xl_hbm.at[src_v.at[0]], buf, sem_a).wait()

        def half_step(j, buf_cur, buf_next):
            wait_gather(buf_cur)
            pltpu.async_copy(buf_cur, agg_sh.at[dst_v.at[j]], sem_a, add=True)

            @pl.when(lax.rem(j, 2) == c)
            def _():
                pltpu.async_copy(ones_v, cnt_sh.at[dst_v.at[j]], sem_c,
                                 add=True)

            @pl.when(j >= 1)
            def _():
                wait_scatter(buf_next)

            @pl.when(j + NBUF - 1 < n_chunks)
            def _():
                start_gather(j + NBUF - 1, buf_next)

        for b in range(NBUF - 1):
            start_gather(b, rows_ring[b])

        def chunk_step(jg, carry):
            for b in range(NBUF):
                half_step(NBUF * jg + b, rows_ring[b],
                          rows_ring[(b + NBUF - 1) % NBUF])
            return carry

        lax.fori_loop(0, n_chunks // NBUF, chunk_step, 0)

        wait_scatter(rows_ring[0])

        def drain_cnt(i, carry):
            pltpu.make_async_copy(ones_hbm, ones_v, sem_c).wait()
            return carry

        lax.fori_loop(0, n_chunks // 2, drain_cnt, 0)
        plsc.subcore_barrier()

        @pl.when(c == 0)
        def _():
            pltpu.sync_copy(agg_sh.at[rslice], agg0_out.at[rslice])
            pltpu.sync_copy(cnt_sh.at[rslice], cnt0_out.at[rslice])

        @pl.when(c == 1)
        def _():
            pltpu.sync_copy(agg_sh.at[rslice], agg1_out.at[rslice])
            pltpu.sync_copy(cnt_sh.at[rslice], cnt1_out.at[rslice])

    return sc_aggregate


def _tc_combine_body(p0, p1, c0, c1, x, wl, wr, b, o):
    dh = p0.shape[1]
    cnt = c0[:, 0:1] + c1[:, 0:1]
    recip = 1.0 / jnp.maximum(cnt, 1.0)
    mean_l = p0[...] * recip
    mean_r = p1[...] * recip
    out = (jnp.dot(mean_l, wl[:dh, :], preferred_element_type=jnp.float32)
           + jnp.dot(mean_r, wl[dh:, :], preferred_element_type=jnp.float32)
           + jnp.dot(x[...], wr[...], preferred_element_type=jnp.float32)
           + b[...])
    o[...] = jnp.maximum(out, 0.0)


def _tc_combine(p0, p1, c0, c1, x, wl, wr, b):
    n, d = x.shape
    dh = d // 2
    blk = 1000
    grid = n // blk
    return pl.pallas_call(
        _tc_combine_body,
        grid=(grid,),
        in_specs=[
            pl.BlockSpec((blk, dh), lambda i: (i, 0)),
            pl.BlockSpec((blk, dh), lambda i: (i, 0)),
            pl.BlockSpec((blk, 16), lambda i: (i, 0)),
            pl.BlockSpec((blk, 16), lambda i: (i, 0)),
            pl.BlockSpec((blk, d), lambda i: (i, 0)),
            pl.BlockSpec((d, d), lambda i: (0, 0)),
            pl.BlockSpec((d, d), lambda i: (0, 0)),
            pl.BlockSpec((1, d), lambda i: (0, 0)),
        ],
        out_specs=pl.BlockSpec((blk, d), lambda i: (i, 0)),
        out_shape=jax.ShapeDtypeStruct((n, d), jnp.float32),
    )(p0, p1, c0, c1, x, wl, wr, b)


def kernel(x, edge_index, W_l, W_r, b):
    n, d = x.shape
    dh = d // 2
    n_edges = edge_index.shape[1]
    edges_per_sub = n_edges // NS
    n_chunks = edges_per_sub // CHUNK
    align = NS * 8
    n_pad = ((n + align - 1) // align) * align
    rows_per_sub = n_pad // NS

    ei = edge_index.astype(jnp.int32)
    src = ei[0].reshape(NS, n_chunks, CHUNK)
    dst = ei[1].reshape(NS, n_chunks, CHUNK)
    xl = x[:, :dh]
    xr = x[:, dh:]
    zag = jnp.zeros((rows_per_sub, dh), jnp.float32)
    zcnt = jnp.zeros((rows_per_sub, 16), jnp.float32)
    ones = jnp.ones((CHUNK, 16), jnp.float32)

    agg0, agg1, cnt0, cnt1 = _make_sc_aggregate(n_pad, dh, n_chunks)(
        xl, xr, src, dst, zag, zcnt, ones)
    return _tc_combine(agg0, agg1, cnt0, cnt1, x, W_l, W_r, b.reshape(1, d))


# speedup vs baseline: 1.5348x; 1.0482x over previous
import functools

import jax
import jax.numpy as jnp
from jax import lax
from jax.experimental import pallas as pl
from jax.experimental.pallas import tpu as pltpu
from jax.experimental.pallas import tpu_sc as plsc

NC = 2
NS = 16
CHUNK = 125
NBUF = 4


def _make_sc_aggregate(n_pad, dh, n_chunks):
    rows_per_sub = n_pad // NS

    @functools.partial(
        pl.kernel,
        out_type=(
            jax.ShapeDtypeStruct((n_pad, dh), jnp.float32),
            jax.ShapeDtypeStruct((n_pad, dh), jnp.float32),
            jax.ShapeDtypeStruct((n_pad, 16), jnp.float32),
            jax.ShapeDtypeStruct((n_pad, 16), jnp.float32),
        ),
        mesh=plsc.VectorSubcoreMesh(
            core_axis_name="c", subcore_axis_name="s",
            num_cores=NC, num_subcores=NS),
        scratch_types=[
            pltpu.VMEM((n_chunks, CHUNK), jnp.int32),
            pltpu.VMEM((n_chunks, CHUNK), jnp.int32),
            [pltpu.VMEM((CHUNK, dh), jnp.float32)] * NBUF,
            pltpu.VMEM((CHUNK, 16), jnp.float32),
            pltpu.VMEM_SHARED((n_pad, dh), jnp.float32),
            pltpu.VMEM_SHARED((n_pad, 16), jnp.float32),
            pltpu.SemaphoreType.DMA,
            pltpu.SemaphoreType.DMA,
            pltpu.SemaphoreType.DMA,
        ],
        compiler_params=pltpu.CompilerParams(use_tc_tiling_on_sc=False),
    )
    def sc_aggregate(xl_hbm, xr_hbm, src_hbm, dst_hbm, zag_hbm, zcnt_hbm,
                     ones_hbm, agg0_out, agg1_out, cnt0_out, cnt1_out,
                     src_v, dst_v, rows_ring, ones_v, agg_sh, cnt_sh,
                     sem_g, sem_a, sem_c):
        c = lax.axis_index("c")
        s = lax.axis_index("s")
        rslice = pl.ds(s * rows_per_sub, rows_per_sub)

        pltpu.sync_copy(zag_hbm, agg_sh.at[rslice])
        pltpu.sync_copy(zcnt_hbm, cnt_sh.at[rslice])
        pltpu.sync_copy(ones_hbm, ones_v)
        pltpu.sync_copy(src_hbm.at[s], src_v)
        pltpu.sync_copy(dst_hbm.at[s], dst_v)
        plsc.subcore_barrier()

        def start_gather(j, buf):
            @pl.when(c == 0)
            def _():
                pltpu.async_copy(xl_hbm.at[src_v.at[j]], buf, sem_g)

            @pl.when(c == 1)
            def _():
                pltpu.async_copy(xr_hbm.at[src_v.at[j]], buf, sem_g)

        def wait_gather(buf):
            pltpu.make_async_copy(xl_hbm.at[src_v.at[0]], buf, sem_g).wait()

        def wait_scatter(buf):
            pltpu.make_async_copy(xl_hbm.at[src_v.at[0]], buf, sem_a).wait()

        def half_step(j, buf_cur, buf_next):
            wait_gather(buf_cur)
            pltpu.async_copy(buf_cur, agg_sh.at[dst_v.at[j]], sem_a, add=True)

            @pl.when(lax.rem(j, 2) == c)
            def _():
                pltpu.async_copy(ones_v, cnt_sh.at[dst_v.at[j]], sem_c,
                                 add=True)

            @pl.when(j >= 1)
            def _():
                wait_scatter(buf_next)

            @pl.when(j + NBUF - 1 < n_chunks)
            def _():
                start_gather(j + NBUF - 1, buf_next)

        for b in range(NBUF - 1):
            start_gather(b, rows_ring[b])

        def chunk_step(jg, carry):
            for b in range(NBUF):
                half_step(NBUF * jg + b, rows_ring[b],
                          rows_ring[(b + NBUF - 1) % NBUF])
            return carry

        lax.fori_loop(0, n_chunks // NBUF, chunk_step, 0)

        wait_scatter(rows_ring[0])

        def drain_cnt(i, carry):
            pltpu.make_async_copy(ones_hbm, ones_v, sem_c).wait()
            return carry

        lax.fori_loop(0, n_chunks // 2, drain_cnt, 0)
        plsc.subcore_barrier()

        @pl.when(c == 0)
        def _():
            pltpu.sync_copy(agg_sh.at[rslice], agg0_out.at[rslice])
            pltpu.sync_copy(cnt_sh.at[rslice], cnt0_out.at[rslice])

        @pl.when(c == 1)
        def _():
            pltpu.sync_copy(agg_sh.at[rslice], agg1_out.at[rslice])
            pltpu.sync_copy(cnt_sh.at[rslice], cnt1_out.at[rslice])

    return sc_aggregate


def _tc_combine_body(p0, p1, c0, c1, x, wl, wr, b, o):
    dh = p0.shape[1]
    cnt = c0[:, 0:1] + c1[:, 0:1]
    recip = 1.0 / jnp.maximum(cnt, 1.0)
    mean_l = p0[...] * recip
    mean_r = p1[...] * recip
    out = (jnp.dot(mean_l, wl[:dh, :], preferred_element_type=jnp.float32)
           + jnp.dot(mean_r, wl[dh:, :], preferred_element_type=jnp.float32)
           + jnp.dot(x[...], wr[...], preferred_element_type=jnp.float32)
           + b[...])
    o[...] = jnp.maximum(out, 0.0)


def _tc_combine(p0, p1, c0, c1, x, wl, wr, b):
    n, d = x.shape
    dh = d // 2
    blk = 1000
    grid = n // blk
    return pl.pallas_call(
        _tc_combine_body,
        grid=(grid,),
        in_specs=[
            pl.BlockSpec((blk, dh), lambda i: (i, 0)),
            pl.BlockSpec((blk, dh), lambda i: (i, 0)),
            pl.BlockSpec((blk, 16), lambda i: (i, 0)),
            pl.BlockSpec((blk, 16), lambda i: (i, 0)),
            pl.BlockSpec((blk, d), lambda i: (i, 0)),
            pl.BlockSpec((d, d), lambda i: (0, 0)),
            pl.BlockSpec((d, d), lambda i: (0, 0)),
            pl.BlockSpec((1, d), lambda i: (0, 0)),
        ],
        out_specs=pl.BlockSpec((blk, d), lambda i: (i, 0)),
        out_shape=jax.ShapeDtypeStruct((n, d), jnp.float32),
    )(p0, p1, c0, c1, x, wl, wr, b)


def kernel(x, edge_index, W_l, W_r, b):
    n, d = x.shape
    dh = d // 2
    n_edges = edge_index.shape[1]
    edges_per_sub = n_edges // NS
    n_chunks = edges_per_sub // CHUNK
    align = NS * 8
    n_pad = ((n + align - 1) // align) * align
    rows_per_sub = n_pad // NS

    ei = edge_index.astype(jnp.int32)
    src = ei[0].reshape(NS, n_chunks, CHUNK)
    dst = ei[1].reshape(NS, n_chunks, CHUNK)
    xl = x[:, :dh]
    xr = x[:, dh:]
    zag = jnp.zeros((rows_per_sub, dh), jnp.float32)
    zcnt = jnp.zeros((rows_per_sub, 16), jnp.float32)
    ones = jnp.ones((CHUNK, 16), jnp.float32)

    agg0, agg1, cnt0, cnt1 = _make_sc_aggregate(n_pad, dh, n_chunks)(
        xl, xr, src, dst, zag, zcnt, ones)
    return _tc_combine(agg0, agg1, cnt0, cnt1, x, W_l, W_r, b.reshape(1, d))
